# baseline (speedup 1.0000x reference)
import functools
import math

import jax
import jax.numpy as jnp
from jax.experimental import pallas as pl
from jax.experimental.pallas import tpu as pltpu

N = 10000
RATIO = 0.8


def _conv_norm(x, src, dst, w, W, b, gamma, beta):
    n = x.shape[0]
    h = x @ W + b
    deg = jnp.zeros((n,), dtype=x.dtype).at[dst].add(w) + 1.0
    norm = w * jax.lax.rsqrt(deg[src]) * jax.lax.rsqrt(deg[dst])
    agg = jnp.zeros_like(h).at[dst].add(norm[:, None] * h[src])
    agg = agg + h / deg[:, None]
    mu = agg.mean(axis=0)
    var = agg.var(axis=0)
    return (agg - mu) * jax.lax.rsqrt(var + 1e-5) * gamma + beta


def _topk_pool(x, src, dst, w, p, ratio):
    n = x.shape[0]
    k = int(math.ceil(ratio * n))
    score = jnp.tanh(x @ p / jnp.linalg.norm(p))
    vals, perm = jax.lax.top_k(score, k)
    x_new = x[perm] * vals[:, None]
    remap = jnp.full((n,), -1, dtype=jnp.int32).at[perm].set(jnp.arange(k, dtype=jnp.int32))
    s2 = remap[src]
    d2 = remap[dst]
    valid = (s2 >= 0) & (d2 >= 0)
    src_new = jnp.where(valid, s2, 0)
    dst_new = jnp.where(valid, d2, 0)
    w_new = jnp.where(valid, w, 0.0)
    return x_new, src_new, dst_new, w_new


def _final_body(h_ref, l2w_ref, l2b_ref, l1w_ref, l1b_ref, l0w_ref, l0b_ref, o_ref):
    h = h_ref[...]
    g = jnp.sum(h, axis=0, keepdims=True) / h.shape[0]
    g = g @ l2w_ref[...] + l2b_ref[...][None, :]
    g = g @ l1w_ref[...] + l1b_ref[...][None, :]
    g = g @ l0w_ref[...] + l0b_ref[...][None, :]
    o_ref[...] = g


def _final_pool_mlp(h, L2W, L2b, L1W, L1b, L0W, L0b):
    out_dim = L0W.shape[1]
    return pl.pallas_call(
        _final_body,
        out_shape=jax.ShapeDtypeStruct((1, out_dim), jnp.float32),
    )(h, L2W, L2b, L1W, L1b, L0W, L0b)


def kernel(x, edge_index, edge_attr, edge_weight, batch,
           W0, b0, g0, be0, W1, b1, g1, be1, W2, b2, g2, be2,
           p0, p1, L0W, L0b, L1W, L1b, L2W, L2b):
    src, dst = edge_index[0], edge_index[1]
    w = edge_weight
    h = jax.nn.relu(_conv_norm(x, src, dst, w, W0, b0, g0, be0))
    h, src, dst, w = _topk_pool(h, src, dst, w, p0, RATIO)
    h = jax.nn.relu(_conv_norm(h, src, dst, w, W1, b1, g1, be1))
    h, src, dst, w = _topk_pool(h, src, dst, w, p1, RATIO)
    h = jax.nn.relu(_conv_norm(h, src, dst, w, W2, b2, g2, be2))
    return _final_pool_mlp(h, L2W, L2b, L1W, L1b, L0W, L0b)


# speedup vs baseline: 4.6077x; 4.6077x over previous
import functools
import math

import jax
import jax.numpy as jnp
from jax import lax
from jax.experimental import pallas as pl
from jax.experimental.pallas import tpu as pltpu
from jax.experimental.pallas import tpu_sc as plsc

N = 10000
E = 320000
NPAD = 10240
E_PAD = 327680
EG = E_PAD // 128
BN = 512
NB = NPAD // BN
NSL = NPAD // 16

NCORES, NSUB, NLANE = 2, 16, 16


def _sc_layer_body(layer, src1, dst1, w1, dst2d, m_h, c_h, h_h,
                   wout, invd_out, s_out, xs_out,
                   m_v, rdeg_v, c_v, src_v, dst_v, w_v, wp_v, iv_v, coef_v,
                   gi_v, dst_i2, dsl_i2, rows_a, rows_b, zv, deg_sv, sem,
                   deg_sh, s_sh, rdeg_sh, xs_sh):
    cid = lax.axis_index("c")
    sid = lax.axis_index("s")
    sl640 = pl.ds(sid * NSL, NSL)
    lane_iota = lax.broadcasted_iota(jnp.int32, (16,), 0)

    pltpu.sync_copy(m_h, m_v)
    pltpu.sync_copy(c_h, c_v)

    def _z16(i, c):
        zv[pl.ds(i * 16, 16)] = jnp.zeros((16,), jnp.float32)
        return c
    lax.fori_loop(0, NSL // 16, _z16, 0)

    def _zrows(l, c):
        for u in range(8):
            rows_b[l, pl.ds(u * 16, 16)] = jnp.zeros((16,), jnp.float32)
        return c
    lax.fori_loop(0, 128, _zrows, 0)

    pltpu.sync_copy(zv, deg_sh.at[sl640])
    pltpu.sync_copy(zv, s_sh.at[sl640])
    for i in range(5):
        pltpu.sync_copy(rows_b.at[pl.ds(0, 64)],
                        xs_sh.at[pl.ds(sid * 320 + i * 64, 64)])
    plsc.subcore_barrier()

    def _phA(mi, c):
        g0 = sid * 160 + mi * 16
        e0 = g0 * 128
        pltpu.sync_copy(src1.at[pl.ds(e0, 2048)], src_v)
        pltpu.sync_copy(dst1.at[pl.ds(e0, 2048)], dst_v)
        pltpu.sync_copy(dst2d.at[pl.ds(g0, 16)], dst_i2)
        pltpu.sync_copy(w1.at[pl.ds(e0, 2048)], w_v)

        def _lane(l, cc):
            sl = pl.ds(l * 16, 16)
            si = src_v[sl]
            di = dst_v[sl]
            ms = plsc.load_gather(m_v, [si])
            md = plsc.load_gather(m_v, [di])
            wp_v[sl] = w_v[sl] * ms * md
            return cc
        lax.fori_loop(0, 128, _lane, 0)

        @pl.when(cid == 0)
        def _():
            pltpu.sync_copy(wp_v, wout.at[pl.ds(e0, 2048)])

        for j in range(16):
            pltpu.sync_copy(wp_v.at[pl.ds(j * 128, 128)],
                            deg_sh.at[dst_i2.at[j]], add=True)
        return c
    lax.fori_loop(0, EG // NSUB // 16, _phA, 0)
    plsc.subcore_barrier()

    pltpu.sync_copy(deg_sh.at[sl640], deg_sv)

    def _newton(i, c):
        sl = pl.ds(i * 16, 16)
        d = deg_sv[sl] + 1.0
        ib = plsc.bitcast(d, jnp.int32)
        y = plsc.bitcast(jnp.int32(0x5F3759DF) - (ib >> 1), jnp.float32)
        for _ in range(4):
            y = y * (1.5 - 0.5 * d * y * y)
        deg_sv[sl] = y
        iv_v[sl] = y * y
        return c
    lax.fori_loop(0, NSL // 16, _newton, 0)
    pltpu.sync_copy(deg_sv, rdeg_sh.at[sl640])

    @pl.when(cid == 0)
    def _():
        pltpu.sync_copy(iv_v, invd_out.at[sl640])
    plsc.subcore_barrier()
    pltpu.sync_copy(rdeg_sh, rdeg_v)

    if layer == 0:
        passes = [(0, dh, 5) for dh in (0, 1)]
    elif layer == 1:
        passes = [(0, dh, 10) for dh in (0, 1)]
    else:
        passes = [(p, dh, 10) for p in (0, 1) for dh in (0, 1)]

    NH = NPAD // 2

    for pi, (p, dh, n_macro) in enumerate(passes):
        if pi > 0:
            lax.fori_loop(0, 128, _zrows, 0)
            for i in range(5):
                pltpu.sync_copy(rows_b.at[pl.ds(0, 64)],
                                xs_sh.at[pl.ds(sid * 320 + i * 64, 64)])
            plsc.subcore_barrier()

        if layer == 0:
            ck = jnp.int32(0)
            base_g = cid * (EG // 2) + sid * 80
        elif layer == 1:
            ck = cid
            base_g = sid * 160
        else:
            ck = cid * 2 + p
            base_g = sid * 160

        def _phC(mi, c, base_g=base_g, ck=ck, p=p, dh=dh):
            g0 = base_g + mi * 16
            e0 = g0 * 128
            pltpu.sync_copy(src1.at[pl.ds(e0, 2048)], src_v)
            pltpu.sync_copy(dst1.at[pl.ds(e0, 2048)], dst_v)
            pltpu.sync_copy(dst2d.at[pl.ds(g0, 16)], dst_i2)
            pltpu.sync_copy(w1.at[pl.ds(e0, 2048)], w_v)

            def _lane(j, cc):
                for u in range(8):
                    sl = pl.ds(j * 128 + u * 16, 16)
                    si = src_v[sl]
                    di = dst_v[sl]
                    ms = plsc.load_gather(m_v, [si])
                    md = plsc.load_gather(m_v, [di])
                    rs = plsc.load_gather(rdeg_v, [si])
                    rd = plsc.load_gather(rdeg_v, [di])
                    cs = plsc.load_gather(c_v, [si])
                    nrm = w_v[sl] * ms * md * rs * rd
                    wp_v[sl] = nrm
                    coef_v[sl] = nrm * cs
                    gi_v[sl] = si + ck * NPAD
                    li = di - dh * NH
                    ok = (li >= 0) & (li < NH)
                    dsl_i2[j, pl.ds(u * 16, 16)] = jnp.where(
                        ok, li, NH + (di & 7))
                return cc
            lax.fori_loop(0, 16, _lane, 0)

            if p == 0 and dh == 0:
                if layer == 0:
                    for j in range(16):
                        pltpu.sync_copy(wp_v.at[pl.ds(j * 128, 128)],
                                        s_sh.at[dst_i2.at[j]], add=True)
                else:
                    @pl.when((sid // 8) == cid)
                    def _():
                        for j in range(16):
                            pltpu.sync_copy(wp_v.at[pl.ds(j * 128, 128)],
                                            s_sh.at[dst_i2.at[j]], add=True)

            for j in range(16):
                pltpu.async_copy(
                    h_h.at[gi_v.at[pl.ds(j * 128, 128)]],
                    rows_a, sem).wait()

                def _row(l, cc, j=j):
                    cf = plsc.load_gather(
                        coef_v, [jnp.zeros((16,), jnp.int32) + (j * 128) + l])
                    for u in range(8):
                        sl = pl.ds(u * 16, 16)
                        rows_b[l, sl] = rows_a[l, sl] * cf
                    return cc
                lax.fori_loop(0, 128, _row, 0)
                pltpu.sync_copy(rows_b, xs_sh.at[dsl_i2.at[j]], add=True)
            return c
        lax.fori_loop(0, n_macro, _phC, 0)
        plsc.subcore_barrier()

        if layer == 2:
            xrow = cid * 2 + p
        else:
            xrow = cid
        off = xrow * NPAD + dh * NH + sid * 320
        for i in range(5):
            pltpu.sync_copy(xs_sh.at[pl.ds(sid * 320 + i * 64, 64)],
                            xs_out.at[pl.ds(off + i * 64, 64)])
        plsc.subcore_barrier()

    pltpu.sync_copy(s_sh.at[sl640],
                    s_out.at[pl.ds(cid * NPAD + sid * NSL, NSL)])


def _make_sc_layer(layer, c_in):
    co = {0: 2, 1: 2, 2: 4}[layer]
    body = functools.partial(_sc_layer_body, layer)
    return pl.kernel(
        body,
        out_type=(
            jax.ShapeDtypeStruct((E_PAD,), jnp.float32),
            jax.ShapeDtypeStruct((NPAD,), jnp.float32),
            jax.ShapeDtypeStruct((2 * NPAD,), jnp.float32),
            jax.ShapeDtypeStruct((co * NPAD, 128), jnp.float32),
        ),
        mesh=plsc.VectorSubcoreMesh(core_axis_name="c", subcore_axis_name="s",
                                    num_cores=NCORES, num_subcores=NSUB),
        compiler_params=pltpu.CompilerParams(needs_layout_passes=False),
        scratch_types=[
            pltpu.VMEM((NPAD,), jnp.float32),
            pltpu.VMEM((NPAD,), jnp.float32),
            pltpu.VMEM((NPAD,), jnp.float32),
            pltpu.VMEM((2048,), jnp.int32),
            pltpu.VMEM((2048,), jnp.int32),
            pltpu.VMEM((2048,), jnp.float32),
            pltpu.VMEM((2048,), jnp.float32),
            pltpu.VMEM((NSL,), jnp.float32),
            pltpu.VMEM((2048,), jnp.float32),
            pltpu.VMEM((2048,), jnp.int32),
            pltpu.VMEM((16, 128), jnp.int32),
            pltpu.VMEM((16, 128), jnp.int32),
            pltpu.VMEM((128, 128), jnp.float32),
            pltpu.VMEM((128, 128), jnp.float32),
            pltpu.VMEM((NSL,), jnp.float32),
            pltpu.VMEM((NSL,), jnp.float32),
            pltpu.SemaphoreType.DMA,
            pltpu.VMEM_SHARED((NPAD,), jnp.float32),
            pltpu.VMEM_SHARED((NPAD,), jnp.float32),
            pltpu.VMEM_SHARED((NPAD,), jnp.float32),
            pltpu.VMEM_SHARED((NPAD // 2 + 8, 128), jnp.float32),
        ],
    )


@functools.lru_cache(maxsize=None)
def _get_sc_layer(layer):
    return _make_sc_layer(layer, {0: 1, 1: 2, 2: 4}[layer])


def _d1_body(P, xs_ref, h_ref, sa_ref, sb_ref, iv_ref, cv_ref, m_ref,
             W_ref, b_ref, agg_ref, sum_ref, sq_ref):
    c = pl.program_id(1)
    C = pl.num_programs(1)
    rb = pl.program_id(0)
    if P == 2:
        t = xs_ref[0] + xs_ref[1]
    else:
        t = xs_ref[0]
    q = cv_ref[0, 0, :] * iv_ref[0, 0, :]
    t = t + q[:, None] * h_ref[0]
    part = jnp.dot(t, W_ref[0], preferred_element_type=jnp.float32)

    @pl.when(c == 0)
    def _():
        agg_ref[...] = part

    @pl.when(c > 0)
    def _():
        agg_ref[...] += part

    @pl.when(c == C - 1)
    def _():
        r = sa_ref[0, 0, :] + sb_ref[0, 0, :] + iv_ref[0, 0, :]
        agg = agg_ref[...] + r[:, None] * b_ref[0]
        agg_ref[...] = agg
        m = m_ref[0, 0, :]
        sm = jnp.sum(m[:, None] * agg, axis=0)[None, None, :]
        sq = jnp.sum(m[:, None] * agg * agg, axis=0)[None, None, :]

        @pl.when(rb == 0)
        def _():
            sum_ref[...] = sm
            sq_ref[...] = sq

        @pl.when(rb > 0)
        def _():
            sum_ref[...] += sm
            sq_ref[...] += sq


def _d1_call(layer, xs, h, sa, sb, iv, cv, m, W, b):
    c_in = {0: 1, 1: 2, 2: 4}[layer]
    P = 2 if layer == 0 else 1
    dout = W.shape[-1]
    if layer == 0:
        xs_spec = pl.BlockSpec((2, BN, 128), lambda rb, c: (0, rb, 0))
    else:
        xs_spec = pl.BlockSpec((1, BN, 128), lambda rb, c: (c, rb, 0))
    vec = pl.BlockSpec((1, 1, BN), lambda rb, c: (rb, 0, 0))
    return pl.pallas_call(
        functools.partial(_d1_body, P),
        grid=(NB, c_in),
        in_specs=[
            xs_spec,
            pl.BlockSpec((1, BN, 128), lambda rb, c: (c, rb, 0)),
            vec, vec, vec, vec, vec,
            pl.BlockSpec((1, 128, dout), lambda rb, c: (c, 0, 0)),
            pl.BlockSpec((1, dout), lambda rb, c: (0, 0)),
        ],
        out_specs=[
            pl.BlockSpec((BN, dout), lambda rb, c: (rb, 0)),
            pl.BlockSpec((1, 1, dout), lambda rb, c: (0, 0, 0)),
            pl.BlockSpec((1, 1, dout), lambda rb, c: (0, 0, 0)),
        ],
        out_shape=[
            jax.ShapeDtypeStruct((NPAD, dout), jnp.float32),
            jax.ShapeDtypeStruct((1, 1, dout), jnp.float32),
            jax.ShapeDtypeStruct((1, 1, dout), jnp.float32),
        ],
    )(xs, h, sa, sb, iv, cv, m, W, b)


def _d2_body(k, agg_ref, sum_ref, sq_ref, g_ref, be_ref, p_ref, h_ref, z_ref):
    c = pl.program_id(1)
    mu = sum_ref[0, 0, :] * (1.0 / k)
    ex2 = sq_ref[0, 0, :] * (1.0 / k)
    var = ex2 - mu * mu
    inv = lax.rsqrt(var + 1e-5)
    hh = (agg_ref[...] - mu[None, :]) * inv[None, :] * g_ref[0, 0, :][None, :] \
        + be_ref[0, 0, :][None, :]
    hh = jnp.maximum(hh, 0.0)
    h_ref[0] = hh
    zp = jnp.dot(hh, p_ref[0, 0, :][:, None],
                 preferred_element_type=jnp.float32)[:, 0]

    @pl.when(c == 0)
    def _():
        z_ref[...] = zp[None, None, :]

    @pl.when(c > 0)
    def _():
        z_ref[...] += zp[None, None, :]


def _d2_call(k, agg, s1, s2, g, be, p):
    dout = agg.shape[1]
    co = dout // 128
    stat = pl.BlockSpec((1, 1, 128), lambda rb, c: (0, 0, c))
    return pl.pallas_call(
        functools.partial(_d2_body, float(k)),
        grid=(NB, co),
        in_specs=[
            pl.BlockSpec((BN, 128), lambda rb, c: (rb, c)),
            stat, stat, stat, stat,
            pl.BlockSpec((1, 1, 128), lambda rb, c: (c, 0, 0)),
        ],
        out_specs=[
            pl.BlockSpec((1, BN, 128), lambda rb, c: (c, rb, 0)),
            pl.BlockSpec((1, 1, BN), lambda rb, c: (rb, 0, 0)),
        ],
        out_shape=[
            jax.ShapeDtypeStruct((co, NPAD, 128), jnp.float32),
            jax.ShapeDtypeStruct((NB, 1, BN), jnp.float32),
        ],
    )(agg, s1, s2, g, be, p)


def _d3_body(k, z_ref, m_ref, p_ref, mn_ref, cv_ref):
    z = z_ref[...]
    m = m_ref[...]
    pv = p_ref[...]
    pn = jnp.sqrt(jnp.sum(pv * pv))
    zi = lax.bitcast_convert_type(z, jnp.int32)
    key = zi ^ ((zi >> 31) & jnp.int32(0x7FFFFFFF))
    ku = lax.bitcast_convert_type(key, jnp.uint32) ^ jnp.uint32(0x80000000)
    ku = jnp.where(m > 0.0, ku, jnp.uint32(0))

    def _bit(i, T):
        cand = T | (jnp.uint32(1) << (jnp.uint32(31) - i.astype(jnp.uint32)))
        cnt = jnp.sum(jnp.where(ku >= cand, jnp.int32(1), jnp.int32(0)))
        return jnp.where(cnt >= k, cand, T)
    T = lax.fori_loop(0, 32, _bit, jnp.uint32(0))

    ngt = jnp.sum(jnp.where(ku > T, jnp.int32(1), jnp.int32(0)))
    mrem = jnp.int32(k) - ngt
    eq = ku == T
    idx = (lax.broadcasted_iota(jnp.int32, z.shape, 0) * BN
           + lax.broadcasted_iota(jnp.int32, z.shape, 2))

    def _bit2(i, Cc):
        cand = Cc | (jnp.int32(1) << (jnp.int32(13) - i))
        f = jnp.sum(jnp.where(eq & (idx < cand), jnp.int32(1), jnp.int32(0)))
        return jnp.where(f < mrem, cand, Cc)
    Cc = lax.fori_loop(0, 14, _bit2, jnp.int32(0))

    sel = (ku > T) | (eq & (idx <= Cc) & (mrem > 0))
    mn = sel.astype(jnp.float32)
    mn_ref[...] = mn
    cv_ref[...] = jnp.tanh(z * (1.0 / pn)) * mn


def _d3_call(k, z, m, p):
    d = p.shape[-1]
    return pl.pallas_call(
        functools.partial(_d3_body, k),
        in_specs=[
            pl.BlockSpec((NB, 1, BN), lambda: (0, 0, 0)),
            pl.BlockSpec((NB, 1, BN), lambda: (0, 0, 0)),
            pl.BlockSpec((1, 1, d), lambda: (0, 0, 0)),
        ],
        out_specs=[
            pl.BlockSpec((NB, 1, BN), lambda: (0, 0, 0)),
            pl.BlockSpec((NB, 1, BN), lambda: (0, 0, 0)),
        ],
        out_shape=[
            jax.ShapeDtypeStruct((NB, 1, BN), jnp.float32),
            jax.ShapeDtypeStruct((NB, 1, BN), jnp.float32),
        ],
    )(z, m, p)


def _f_body(k2, h_ref, m_ref, w2_ref, b2_ref, w1_ref, b1_ref, w0_ref, b0_ref,
            o_ref, acc_ref):
    rb = pl.program_id(0)
    m = m_ref[0, 0, :]
    s = jnp.sum(h_ref[...] * m[None, :, None], axis=1)

    @pl.when(rb == 0)
    def _():
        acc_ref[...] = s

    @pl.when(rb > 0)
    def _():
        acc_ref[...] += s

    @pl.when(rb == NB - 1)
    def _():
        a = acc_ref[...] * (1.0 / k2)
        g = jnp.zeros((1, w2_ref.shape[1]), jnp.float32)
        for j in range(8):
            g = g + jnp.dot(a[j][None, :], w2_ref[pl.ds(j * 128, 128), :],
                            preferred_element_type=jnp.float32)
        g = g + b2_ref[...]
        g = jnp.dot(g, w1_ref[...], preferred_element_type=jnp.float32) \
            + b1_ref[...]
        g = jnp.dot(g, w0_ref[...], preferred_element_type=jnp.float32) \
            + b0_ref[...]
        o_ref[...] = g


def _f_call(k2, h2, m2, L2W, L2b, L1W, L1b, L0W, L0b):
    return pl.pallas_call(
        functools.partial(_f_body, float(k2)),
        grid=(NB,),
        in_specs=[
            pl.BlockSpec((8, BN, 128), lambda rb: (0, rb, 0)),
            pl.BlockSpec((1, 1, BN), lambda rb: (rb, 0, 0)),
            pl.BlockSpec((1024, 512), lambda rb: (0, 0)),
            pl.BlockSpec((1, 512), lambda rb: (0, 0)),
            pl.BlockSpec((512, 256), lambda rb: (0, 0)),
            pl.BlockSpec((1, 256), lambda rb: (0, 0)),
            pl.BlockSpec((256, 32), lambda rb: (0, 0)),
            pl.BlockSpec((1, 32), lambda rb: (0, 0)),
        ],
        out_specs=pl.BlockSpec((1, 32), lambda rb: (0, 0)),
        out_shape=jax.ShapeDtypeStruct((1, 32), jnp.float32),
        scratch_shapes=[pltpu.VMEM((8, 128), jnp.float32)],
    )(h2, m2, L2W, L2b, L1W, L1b, L0W, L0b)


def kernel(x, edge_index, edge_attr, edge_weight, batch,
           W0, b0, g0, be0, W1, b1, g1, be1, W2, b2, g2, be2,
           p0, p1, L0W, L0b, L1W, L1b, L2W, L2b):
    f32 = jnp.float32
    src = edge_index[0]
    dst = edge_index[1]
    epad = E_PAD - E
    pad_idx = (jnp.arange(epad, dtype=jnp.int32) % N)
    src1 = jnp.concatenate([src, pad_idx])
    dst1 = jnp.concatenate([dst, pad_idx])
    w1 = jnp.concatenate([edge_weight, jnp.zeros((epad,), f32)])
    dst2d = dst1.reshape(EG, 128)
    dst1 = jnp.concatenate([dst1, jnp.zeros((8,), jnp.int32)])

    mask0 = (jnp.arange(NPAD) < N).astype(f32)
    m3 = mask0.reshape(NB, 1, BN)
    c0 = jnp.where(jnp.arange(NPAD) < N, 1.0, 0.0).astype(f32)
    x_p = jnp.pad(x, ((0, NPAD - N), (0, 0)))

    ks = [10000, 8000, 6400]
    Ws = [W0.reshape(1, 128, 256), W1.reshape(2, 128, 512),
          W2.reshape(4, 128, 1024)]
    bs = [b0.reshape(1, 256), b1.reshape(1, 512), b2.reshape(1, 1024)]
    gs = [g0.reshape(1, 1, 256), g1.reshape(1, 1, 512), g2.reshape(1, 1, 1024)]
    bes = [be0.reshape(1, 1, 256), be1.reshape(1, 1, 512),
           be2.reshape(1, 1, 1024)]
    ps = [p0, p1, None]

    m_flat = mask0
    c_flat = c0
    h_flat = x_p
    w_cur = w1
    z = None
    for layer in range(3):
        wout, invd, s2, xs = _get_sc_layer(layer)(
            src1, dst1, w_cur, dst2d, m_flat, c_flat, h_flat)
        c_in = {0: 1, 1: 2, 2: 4}[layer]
        co = {0: 2, 1: 2, 2: 4}[layer]
        xs3 = xs.reshape(co, NPAD, 128)
        h3 = h_flat.reshape(c_in, NPAD, 128)
        sa = s2[:NPAD].reshape(NB, 1, BN)
        sb = s2[NPAD:].reshape(NB, 1, BN)
        iv = invd.reshape(NB, 1, BN)
        cv3 = c_flat.reshape(NB, 1, BN)
        agg, s_sum, s_sq = _d1_call(layer, xs3, h3, sa, sb, iv, cv3, m3,
                                    Ws[layer], bs[layer])
        dout = Ws[layer].shape[2]
        pz = (ps[layer] if ps[layer] is not None
              else jnp.zeros((dout,), f32)).reshape(dout // 128, 1, 128)
        h_c, z = _d2_call(ks[layer], agg, s_sum, s_sq, gs[layer], bes[layer],
                          pz)
        if layer < 2:
            mn, cv = _d3_call(ks[layer + 1], z, m3,
                              ps[layer].reshape(1, 1, dout))
            m3 = mn
            m_flat = mn.reshape(NPAD)
            c_flat = cv.reshape(NPAD)
            h_flat = h_c.reshape(-1, 128)
            w_cur = wout
        else:
            h2 = h_c

    return _f_call(ks[2], h2, m3, L2W, L2b.reshape(1, 512),
                   L1W, L1b.reshape(1, 256), L0W, L0b.reshape(1, 32))


# speedup vs baseline: 11.7475x; 2.5495x over previous
import functools
import math

import jax
import jax.numpy as jnp
from jax import lax
from jax.experimental import pallas as pl
from jax.experimental.pallas import tpu as pltpu
from jax.experimental.pallas import tpu_sc as plsc

N = 10000
E = 320000
NPAD = 10240
E_PAD = 327680
EG = E_PAD // 128
BN = 512
NB = NPAD // BN
NSL = NPAD // 16

NCORES, NSUB, NLANE = 2, 16, 16


def _sc_layer_body(layer, src1, dst1, w1, dst2d, m_h, c_h, h_h,
                   wout, invd_out, s_out, xs_out,
                   m_v, rdeg_v, c_v, src_v, dst_v, w_v, wp_v, iv_v, coef_v,
                   gi_v, dst_i2, dsl_i2, rows_a, rows_b, zv, deg_sv,
                   sem, sem2, deg_sh, s_sh, rdeg_sh, xs_sh):
    cid = lax.axis_index("c")
    sid = lax.axis_index("s")
    sl640 = pl.ds(sid * NSL, NSL)
    lane_iota = lax.broadcasted_iota(jnp.int32, (16,), 0)

    pltpu.sync_copy(m_h, m_v)
    pltpu.sync_copy(c_h, c_v)

    def _z16(i, c):
        zv[pl.ds(i * 16, 16)] = jnp.zeros((16,), jnp.float32)
        return c
    lax.fori_loop(0, NSL // 16, _z16, 0)

    def _zrows(l, c):
        for u in range(8):
            rows_b[l, pl.ds(u * 16, 16)] = jnp.zeros((16,), jnp.float32)
        return c
    lax.fori_loop(0, 128, _zrows, 0)

    pltpu.sync_copy(zv, deg_sh.at[sl640])
    pltpu.sync_copy(zv, s_sh.at[sl640])
    for i in range(5):
        pltpu.sync_copy(rows_b.at[pl.ds(0, 64)],
                        xs_sh.at[pl.ds(sid * 320 + i * 64, 64)])
    plsc.subcore_barrier()

    def _phA(mi, c):
        g0 = sid * 160 + mi * 16
        e0 = g0 * 128
        pltpu.sync_copy(src1.at[pl.ds(e0, 2048)], src_v)
        pltpu.sync_copy(dst1.at[pl.ds(e0, 2048)], dst_v)
        pltpu.sync_copy(dst2d.at[pl.ds(g0, 16)], dst_i2)
        pltpu.sync_copy(w1.at[pl.ds(e0, 2048)], w_v)

        def _lane(l, cc):
            sl = pl.ds(l * 16, 16)
            si = src_v[sl]
            di = dst_v[sl]
            ms = plsc.load_gather(m_v, [si])
            md = plsc.load_gather(m_v, [di])
            wp_v[sl] = w_v[sl] * ms * md
            return cc
        lax.fori_loop(0, 128, _lane, 0)

        @pl.when(cid == 0)
        def _():
            pltpu.sync_copy(wp_v, wout.at[pl.ds(e0, 2048)])

        def _dsc(j, cc):
            pltpu.sync_copy(wp_v.at[pl.ds(j * 128, 128)],
                            deg_sh.at[dst_i2.at[j]], add=True)
            return cc
        lax.fori_loop(0, 16, _dsc, 0)
        return c
    lax.fori_loop(0, EG // NSUB // 16, _phA, 0)
    plsc.subcore_barrier()

    pltpu.sync_copy(deg_sh.at[sl640], deg_sv)

    def _newton(i, c):
        sl = pl.ds(i * 16, 16)
        d = deg_sv[sl] + 1.0
        ib = plsc.bitcast(d, jnp.int32)
        y = plsc.bitcast(jnp.int32(0x5F3759DF) - (ib >> 1), jnp.float32)
        for _ in range(4):
            y = y * (1.5 - 0.5 * d * y * y)
        deg_sv[sl] = y
        iv_v[sl] = y * y
        return c
    lax.fori_loop(0, NSL // 16, _newton, 0)
    pltpu.sync_copy(deg_sv, rdeg_sh.at[sl640])

    @pl.when(cid == 0)
    def _():
        pltpu.sync_copy(iv_v, invd_out.at[sl640])
    plsc.subcore_barrier()
    pltpu.sync_copy(rdeg_sh, rdeg_v)

    if layer == 0:
        passes = [(0, dh, 5) for dh in (0, 1)]
    elif layer == 1:
        passes = [(0, dh, 10) for dh in (0, 1)]
    else:
        passes = [(p, dh, 10) for p in (0, 1) for dh in (0, 1)]

    NH = NPAD // 2

    for pi, (p, dh, n_macro) in enumerate(passes):
        if pi > 0:
            lax.fori_loop(0, 128, _zrows, 0)
            for i in range(5):
                pltpu.sync_copy(rows_b.at[pl.ds(0, 64)],
                                xs_sh.at[pl.ds(sid * 320 + i * 64, 64)])
            plsc.subcore_barrier()

        if layer == 0:
            ck = jnp.int32(0)
            base_g = cid * (EG // 2) + sid * 80
        elif layer == 1:
            ck = cid
            base_g = sid * 160
        else:
            ck = cid * 2 + p
            base_g = sid * 160

        def _phC(mi, c, base_g=base_g, ck=ck, p=p, dh=dh):
            g0 = base_g + mi * 16
            e0 = g0 * 128
            pltpu.sync_copy(src1.at[pl.ds(e0, 2048)], src_v)
            pltpu.sync_copy(dst1.at[pl.ds(e0, 2048)], dst_v)
            pltpu.sync_copy(dst2d.at[pl.ds(g0, 16)], dst_i2)
            pltpu.sync_copy(w1.at[pl.ds(e0, 2048)], w_v)

            def _lane(j, cc):
                for u in range(8):
                    sl = pl.ds(j * 128 + u * 16, 16)
                    si = src_v[sl]
                    di = dst_v[sl]
                    ms = plsc.load_gather(m_v, [si])
                    md = plsc.load_gather(m_v, [di])
                    rs = plsc.load_gather(rdeg_v, [si])
                    rd = plsc.load_gather(rdeg_v, [di])
                    cs = plsc.load_gather(c_v, [si])
                    nrm = w_v[sl] * ms * md * rs * rd
                    wp_v[sl] = nrm
                    coef_v[sl] = nrm * cs
                    gi_v[sl] = si + ck * NPAD
                    li = di - dh * NH
                    ok = (li >= 0) & (li < NH)
                    dsl_i2[j, pl.ds(u * 16, 16)] = jnp.where(
                        ok, li, NH + (di & 7))
                return cc
            lax.fori_loop(0, 16, _lane, 0)

            if p == 0 and dh == 0:
                def _ssc(j, cc):
                    pltpu.sync_copy(wp_v.at[pl.ds(j * 128, 128)],
                                    s_sh.at[dst_i2.at[j]], add=True)
                    return cc
                if layer == 0:
                    lax.fori_loop(0, 16, _ssc, 0)
                else:
                    @pl.when((sid // 8) == cid)
                    def _():
                        lax.fori_loop(0, 16, _ssc, 0)

            def _gather(j, buf, s):
                return pltpu.async_copy(
                    h_h.at[gi_v.at[pl.ds(j * 128, 128)]], buf, s)

            def _process(j, buf):
                def _row(l, cc):
                    cf = plsc.load_gather(
                        coef_v, [jnp.zeros((16,), jnp.int32) + j * 128 + l])
                    for u in range(8):
                        sl = pl.ds(u * 16, 16)
                        buf[l, sl] = buf[l, sl] * cf
                    return cc
                lax.fori_loop(0, 128, _row, 0)
                pltpu.sync_copy(buf, xs_sh.at[dsl_i2.at[j]], add=True)

            _gather(0, rows_a, sem)

            def _pair(jj, cc):
                j0 = jj * 2
                j1 = j0 + 1
                _gather(j1, rows_b, sem2)
                pltpu.make_async_copy(
                    h_h.at[gi_v.at[pl.ds(j0 * 128, 128)]], rows_a, sem).wait()
                _process(j0, rows_a)

                @pl.when(jj < 7)
                def _():
                    _gather(j0 + 2, rows_a, sem)
                pltpu.make_async_copy(
                    h_h.at[gi_v.at[pl.ds(j1 * 128, 128)]], rows_b,
                    sem2).wait()
                _process(j1, rows_b)
                return cc
            lax.fori_loop(0, 8, _pair, 0)
            return c
        lax.fori_loop(0, n_macro, _phC, 0)
        plsc.subcore_barrier()

        if layer == 2:
            xrow = cid * 2 + p
        else:
            xrow = cid
        off = xrow * NPAD + dh * NH + sid * 320
        for i in range(5):
            pltpu.sync_copy(xs_sh.at[pl.ds(sid * 320 + i * 64, 64)],
                            xs_out.at[pl.ds(off + i * 64, 64)])
        plsc.subcore_barrier()

    pltpu.sync_copy(s_sh.at[sl640],
                    s_out.at[pl.ds(cid * NPAD + sid * NSL, NSL)])


def _make_sc_layer(layer, c_in):
    co = {0: 2, 1: 2, 2: 4}[layer]
    body = functools.partial(_sc_layer_body, layer)
    return pl.kernel(
        body,
        out_type=(
            jax.ShapeDtypeStruct((E_PAD,), jnp.float32),
            jax.ShapeDtypeStruct((NPAD,), jnp.float32),
            jax.ShapeDtypeStruct((2 * NPAD,), jnp.float32),
            jax.ShapeDtypeStruct((co * NPAD, 128), jnp.float32),
        ),
        mesh=plsc.VectorSubcoreMesh(core_axis_name="c", subcore_axis_name="s",
                                    num_cores=NCORES, num_subcores=NSUB),
        compiler_params=pltpu.CompilerParams(needs_layout_passes=False),
        scratch_types=[
            pltpu.VMEM((NPAD,), jnp.float32),
            pltpu.VMEM((NPAD,), jnp.float32),
            pltpu.VMEM((NPAD,), jnp.float32),
            pltpu.VMEM((2048,), jnp.int32),
            pltpu.VMEM((2048,), jnp.int32),
            pltpu.VMEM((2048,), jnp.float32),
            pltpu.VMEM((2048,), jnp.float32),
            pltpu.VMEM((NSL,), jnp.float32),
            pltpu.VMEM((2048,), jnp.float32),
            pltpu.VMEM((2048,), jnp.int32),
            pltpu.VMEM((16, 128), jnp.int32),
            pltpu.VMEM((16, 128), jnp.int32),
            pltpu.VMEM((128, 128), jnp.float32),
            pltpu.VMEM((128, 128), jnp.float32),
            pltpu.VMEM((NSL,), jnp.float32),
            pltpu.VMEM((NSL,), jnp.float32),
            pltpu.SemaphoreType.DMA,
            pltpu.SemaphoreType.DMA,
            pltpu.VMEM_SHARED((NPAD,), jnp.float32),
            pltpu.VMEM_SHARED((NPAD,), jnp.float32),
            pltpu.VMEM_SHARED((NPAD,), jnp.float32),
            pltpu.VMEM_SHARED((NPAD // 2 + 8, 128), jnp.float32),
        ],
    )


@functools.lru_cache(maxsize=None)
def _get_sc_layer(layer):
    return _make_sc_layer(layer, {0: 1, 1: 2, 2: 4}[layer])


def _d1_body(P, xs_ref, h_ref, sa_ref, sb_ref, iv_ref, cv_ref, m_ref,
             W_ref, b_ref, agg_ref, sum_ref, sq_ref):
    c = pl.program_id(1)
    C = pl.num_programs(1)
    rb = pl.program_id(0)
    if P == 2:
        t = xs_ref[0] + xs_ref[1]
    else:
        t = xs_ref[0]
    q = cv_ref[0, 0, :] * iv_ref[0, 0, :]
    t = t + q[:, None] * h_ref[0]
    part = jnp.dot(t, W_ref[0], preferred_element_type=jnp.float32)

    @pl.when(c == 0)
    def _():
        agg_ref[...] = part

    @pl.when(c > 0)
    def _():
        agg_ref[...] += part

    @pl.when(c == C - 1)
    def _():
        r = sa_ref[0, 0, :] + sb_ref[0, 0, :] + iv_ref[0, 0, :]
        agg = agg_ref[...] + r[:, None] * b_ref[0]
        agg_ref[...] = agg
        m = m_ref[0, 0, :]
        sm = jnp.sum(m[:, None] * agg, axis=0)[None, None, :]
        sq = jnp.sum(m[:, None] * agg * agg, axis=0)[None, None, :]

        @pl.when(rb == 0)
        def _():
            sum_ref[...] = sm
            sq_ref[...] = sq

        @pl.when(rb > 0)
        def _():
            sum_ref[...] += sm
            sq_ref[...] += sq


def _d1_call(layer, xs, h, sa, sb, iv, cv, m, W, b):
    c_in = {0: 1, 1: 2, 2: 4}[layer]
    P = 2 if layer == 0 else 1
    dout = W.shape[-1]
    if layer == 0:
        xs_spec = pl.BlockSpec((2, BN, 128), lambda rb, c: (0, rb, 0))
    else:
        xs_spec = pl.BlockSpec((1, BN, 128), lambda rb, c: (c, rb, 0))
    vec = pl.BlockSpec((1, 1, BN), lambda rb, c: (rb, 0, 0))
    return pl.pallas_call(
        functools.partial(_d1_body, P),
        grid=(NB, c_in),
        in_specs=[
            xs_spec,
            pl.BlockSpec((1, BN, 128), lambda rb, c: (c, rb, 0)),
            vec, vec, vec, vec, vec,
            pl.BlockSpec((1, 128, dout), lambda rb, c: (c, 0, 0)),
            pl.BlockSpec((1, dout), lambda rb, c: (0, 0)),
        ],
        out_specs=[
            pl.BlockSpec((BN, dout), lambda rb, c: (rb, 0)),
            pl.BlockSpec((1, 1, dout), lambda rb, c: (0, 0, 0)),
            pl.BlockSpec((1, 1, dout), lambda rb, c: (0, 0, 0)),
        ],
        out_shape=[
            jax.ShapeDtypeStruct((NPAD, dout), jnp.float32),
            jax.ShapeDtypeStruct((1, 1, dout), jnp.float32),
            jax.ShapeDtypeStruct((1, 1, dout), jnp.float32),
        ],
    )(xs, h, sa, sb, iv, cv, m, W, b)


def _d2_body(k, agg_ref, sum_ref, sq_ref, g_ref, be_ref, p_ref, h_ref, z_ref):
    c = pl.program_id(1)
    mu = sum_ref[0, 0, :] * (1.0 / k)
    ex2 = sq_ref[0, 0, :] * (1.0 / k)
    var = ex2 - mu * mu
    inv = lax.rsqrt(var + 1e-5)
    hh = (agg_ref[...] - mu[None, :]) * inv[None, :] * g_ref[0, 0, :][None, :] \
        + be_ref[0, 0, :][None, :]
    hh = jnp.maximum(hh, 0.0)
    h_ref[0] = hh
    zp = jnp.dot(hh, p_ref[0, 0, :][:, None],
                 preferred_element_type=jnp.float32)[:, 0]

    @pl.when(c == 0)
    def _():
        z_ref[...] = zp[None, None, :]

    @pl.when(c > 0)
    def _():
        z_ref[...] += zp[None, None, :]


def _d2_call(k, agg, s1, s2, g, be, p):
    dout = agg.shape[1]
    co = dout // 128
    stat = pl.BlockSpec((1, 1, 128), lambda rb, c: (0, 0, c))
    return pl.pallas_call(
        functools.partial(_d2_body, float(k)),
        grid=(NB, co),
        in_specs=[
            pl.BlockSpec((BN, 128), lambda rb, c: (rb, c)),
            stat, stat, stat, stat,
            pl.BlockSpec((1, 1, 128), lambda rb, c: (c, 0, 0)),
        ],
        out_specs=[
            pl.BlockSpec((1, BN, 128), lambda rb, c: (c, rb, 0)),
            pl.BlockSpec((1, 1, BN), lambda rb, c: (rb, 0, 0)),
        ],
        out_shape=[
            jax.ShapeDtypeStruct((co, NPAD, 128), jnp.float32),
            jax.ShapeDtypeStruct((NB, 1, BN), jnp.float32),
        ],
    )(agg, s1, s2, g, be, p)


def _d3_body(k, z_ref, m_ref, p_ref, mn_ref, cv_ref):
    z = z_ref[...]
    m = m_ref[...]
    pv = p_ref[...]
    pn = jnp.sqrt(jnp.sum(pv * pv))
    zi = lax.bitcast_convert_type(z, jnp.int32)
    key = zi ^ ((zi >> 31) & jnp.int32(0x7FFFFFFF))
    ku = lax.bitcast_convert_type(key, jnp.uint32) ^ jnp.uint32(0x80000000)
    ku = jnp.where(m > 0.0, ku, jnp.uint32(0))

    def _bit(i, T):
        cand = T | (jnp.uint32(1) << (jnp.uint32(31) - i.astype(jnp.uint32)))
        cnt = jnp.sum(jnp.where(ku >= cand, jnp.int32(1), jnp.int32(0)))
        return jnp.where(cnt >= k, cand, T)
    T = lax.fori_loop(0, 32, _bit, jnp.uint32(0))

    ngt = jnp.sum(jnp.where(ku > T, jnp.int32(1), jnp.int32(0)))
    mrem = jnp.int32(k) - ngt
    eq = ku == T
    idx = (lax.broadcasted_iota(jnp.int32, z.shape, 0) * BN
           + lax.broadcasted_iota(jnp.int32, z.shape, 2))

    def _bit2(i, Cc):
        cand = Cc | (jnp.int32(1) << (jnp.int32(13) - i))
        f = jnp.sum(jnp.where(eq & (idx < cand), jnp.int32(1), jnp.int32(0)))
        return jnp.where(f < mrem, cand, Cc)
    Cc = lax.fori_loop(0, 14, _bit2, jnp.int32(0))

    sel = (ku > T) | (eq & (idx <= Cc) & (mrem > 0))
    mn = sel.astype(jnp.float32)
    mn_ref[...] = mn
    cv_ref[...] = jnp.tanh(z * (1.0 / pn)) * mn


def _d3_call(k, z, m, p):
    d = p.shape[-1]
    return pl.pallas_call(
        functools.partial(_d3_body, k),
        in_specs=[
            pl.BlockSpec((NB, 1, BN), lambda: (0, 0, 0)),
            pl.BlockSpec((NB, 1, BN), lambda: (0, 0, 0)),
            pl.BlockSpec((1, 1, d), lambda: (0, 0, 0)),
        ],
        out_specs=[
            pl.BlockSpec((NB, 1, BN), lambda: (0, 0, 0)),
            pl.BlockSpec((NB, 1, BN), lambda: (0, 0, 0)),
        ],
        out_shape=[
            jax.ShapeDtypeStruct((NB, 1, BN), jnp.float32),
            jax.ShapeDtypeStruct((NB, 1, BN), jnp.float32),
        ],
    )(z, m, p)


def _f_body(k2, h_ref, m_ref, w2_ref, b2_ref, w1_ref, b1_ref, w0_ref, b0_ref,
            o_ref, acc_ref):
    rb = pl.program_id(0)
    m = m_ref[0, 0, :]
    s = jnp.sum(h_ref[...] * m[None, :, None], axis=1)

    @pl.when(rb == 0)
    def _():
        acc_ref[...] = s

    @pl.when(rb > 0)
    def _():
        acc_ref[...] += s

    @pl.when(rb == NB - 1)
    def _():
        a = acc_ref[...] * (1.0 / k2)
        g = jnp.zeros((1, w2_ref.shape[1]), jnp.float32)
        for j in range(8):
            g = g + jnp.dot(a[j][None, :], w2_ref[pl.ds(j * 128, 128), :],
                            preferred_element_type=jnp.float32)
        g = g + b2_ref[...]
        g = jnp.dot(g, w1_ref[...], preferred_element_type=jnp.float32) \
            + b1_ref[...]
        g = jnp.dot(g, w0_ref[...], preferred_element_type=jnp.float32) \
            + b0_ref[...]
        o_ref[...] = g


def _f_call(k2, h2, m2, L2W, L2b, L1W, L1b, L0W, L0b):
    return pl.pallas_call(
        functools.partial(_f_body, float(k2)),
        grid=(NB,),
        in_specs=[
            pl.BlockSpec((8, BN, 128), lambda rb: (0, rb, 0)),
            pl.BlockSpec((1, 1, BN), lambda rb: (rb, 0, 0)),
            pl.BlockSpec((1024, 512), lambda rb: (0, 0)),
            pl.BlockSpec((1, 512), lambda rb: (0, 0)),
            pl.BlockSpec((512, 256), lambda rb: (0, 0)),
            pl.BlockSpec((1, 256), lambda rb: (0, 0)),
            pl.BlockSpec((256, 32), lambda rb: (0, 0)),
            pl.BlockSpec((1, 32), lambda rb: (0, 0)),
        ],
        out_specs=pl.BlockSpec((1, 32), lambda rb: (0, 0)),
        out_shape=jax.ShapeDtypeStruct((1, 32), jnp.float32),
        scratch_shapes=[pltpu.VMEM((8, 128), jnp.float32)],
    )(h2, m2, L2W, L2b, L1W, L1b, L0W, L0b)


def kernel(x, edge_index, edge_attr, edge_weight, batch,
           W0, b0, g0, be0, W1, b1, g1, be1, W2, b2, g2, be2,
           p0, p1, L0W, L0b, L1W, L1b, L2W, L2b):
    f32 = jnp.float32
    src = edge_index[0]
    dst = edge_index[1]
    epad = E_PAD - E
    pad_idx = (jnp.arange(epad, dtype=jnp.int32) % N)
    src1 = jnp.concatenate([src, pad_idx])
    dst1 = jnp.concatenate([dst, pad_idx])
    w1 = jnp.concatenate([edge_weight, jnp.zeros((epad,), f32)])
    dst2d = dst1.reshape(EG, 128)
    dst1 = jnp.concatenate([dst1, jnp.zeros((8,), jnp.int32)])

    mask0 = (jnp.arange(NPAD) < N).astype(f32)
    m3 = mask0.reshape(NB, 1, BN)
    c0 = jnp.where(jnp.arange(NPAD) < N, 1.0, 0.0).astype(f32)
    x_p = jnp.pad(x, ((0, NPAD - N), (0, 0)))

    ks = [10000, 8000, 6400]
    Ws = [W0.reshape(1, 128, 256), W1.reshape(2, 128, 512),
          W2.reshape(4, 128, 1024)]
    bs = [b0.reshape(1, 256), b1.reshape(1, 512), b2.reshape(1, 1024)]
    gs = [g0.reshape(1, 1, 256), g1.reshape(1, 1, 512), g2.reshape(1, 1, 1024)]
    bes = [be0.reshape(1, 1, 256), be1.reshape(1, 1, 512),
           be2.reshape(1, 1, 1024)]
    ps = [p0, p1, None]

    m_flat = mask0
    c_flat = c0
    h_flat = x_p
    w_cur = w1
    z = None
    for layer in range(3):
        wout, invd, s2, xs = _get_sc_layer(layer)(
            src1, dst1, w_cur, dst2d, m_flat, c_flat, h_flat)
        c_in = {0: 1, 1: 2, 2: 4}[layer]
        co = {0: 2, 1: 2, 2: 4}[layer]
        xs3 = xs.reshape(co, NPAD, 128)
        h3 = h_flat.reshape(c_in, NPAD, 128)
        sa = s2[:NPAD].reshape(NB, 1, BN)
        sb = s2[NPAD:].reshape(NB, 1, BN)
        iv = invd.reshape(NB, 1, BN)
        cv3 = c_flat.reshape(NB, 1, BN)
        agg, s_sum, s_sq = _d1_call(layer, xs3, h3, sa, sb, iv, cv3, m3,
                                    Ws[layer], bs[layer])
        dout = Ws[layer].shape[2]
        pz = (ps[layer] if ps[layer] is not None
              else jnp.zeros((dout,), f32)).reshape(dout // 128, 1, 128)
        h_c, z = _d2_call(ks[layer], agg, s_sum, s_sq, gs[layer], bes[layer],
                          pz)
        if layer < 2:
            mn, cv = _d3_call(ks[layer + 1], z, m3,
                              ps[layer].reshape(1, 1, dout))
            m3 = mn
            m_flat = mn.reshape(NPAD)
            c_flat = cv.reshape(NPAD)
            h_flat = h_c.reshape(-1, 128)
            w_cur = wout
        else:
            h2 = h_c

    return _f_call(ks[2], h2, m3, L2W, L2b.reshape(1, 512),
                   L1W, L1b.reshape(1, 256), L0W, L0b.reshape(1, 32))


# speedup vs baseline: 12.2221x; 1.0404x over previous
import functools
import math

import jax
import jax.numpy as jnp
from jax import lax
from jax.experimental import pallas as pl
from jax.experimental.pallas import tpu as pltpu
from jax.experimental.pallas import tpu_sc as plsc

N = 10000
E = 320000
NPAD = 10240
E_PAD = 327680
EG = E_PAD // 128
BN = 512
NB = NPAD // BN
NSL = NPAD // 16

NCORES, NSUB, NLANE = 2, 16, 16


def _sc_layer_body(layer, src1, dst1, w1, dst2d, m_h, c_h, h_h,
                   wout, invd_out, s_out, xs_out,
                   m_v, rdeg_v, c_v, src_v, dst_v, w_v, wp_v, iv_v, coef_v,
                   gi_v, dst_i2, dsl_i2, rows_a, rows_b, zv, deg_sv,
                   sem, sem2, deg_sh, s_sh, rdeg_sh, xs_sh):
    cid = lax.axis_index("c")
    sid = lax.axis_index("s")
    sl640 = pl.ds(sid * NSL, NSL)
    lane_iota = lax.broadcasted_iota(jnp.int32, (16,), 0)

    pltpu.sync_copy(m_h, m_v)
    pltpu.sync_copy(c_h, c_v)

    def _z16(i, c):
        zv[pl.ds(i * 16, 16)] = jnp.zeros((16,), jnp.float32)
        return c
    lax.fori_loop(0, NSL // 16, _z16, 0)

    def _zrows(l, c):
        for u in range(8):
            rows_b[l, pl.ds(u * 16, 16)] = jnp.zeros((16,), jnp.float32)
        return c
    lax.fori_loop(0, 128, _zrows, 0)

    pltpu.sync_copy(zv, deg_sh.at[sl640])
    pltpu.sync_copy(zv, s_sh.at[sl640])
    for i in range(5):
        pltpu.sync_copy(rows_b.at[pl.ds(0, 64)],
                        xs_sh.at[pl.ds(sid * 320 + i * 64, 64)])
    plsc.subcore_barrier()

    def _phA(mi, c):
        g0 = sid * 160 + mi * 16
        e0 = g0 * 128
        pltpu.sync_copy(src1.at[pl.ds(e0, 2048)], src_v)
        pltpu.sync_copy(dst1.at[pl.ds(e0, 2048)], dst_v)
        pltpu.sync_copy(dst2d.at[pl.ds(g0, 16)], dst_i2)
        pltpu.sync_copy(w1.at[pl.ds(e0, 2048)], w_v)

        if layer != 0:
            def _lane(l, cc):
                sl = pl.ds(l * 16, 16)
                si = src_v[sl]
                di = dst_v[sl]
                ms = plsc.load_gather(m_v, [si])
                md = plsc.load_gather(m_v, [di])
                wp_v[sl] = w_v[sl] * ms * md
                return cc
            lax.fori_loop(0, 128, _lane, 0)

            @pl.when(cid == 0)
            def _():
                pltpu.sync_copy(wp_v, wout.at[pl.ds(e0, 2048)])
        else:
            @pl.when(cid == 0)
            def _():
                pltpu.sync_copy(w_v, wout.at[pl.ds(e0, 2048)])

        wsrc = w_v if layer == 0 else wp_v

        def _dsc(j, cc):
            pltpu.sync_copy(wsrc.at[pl.ds(j * 128, 128)],
                            deg_sh.at[dst_i2.at[j]], add=True)
            return cc
        lax.fori_loop(0, 16, _dsc, 0)
        return c
    lax.fori_loop(0, EG // NSUB // 16, _phA, 0)
    plsc.subcore_barrier()

    pltpu.sync_copy(deg_sh.at[sl640], deg_sv)

    def _newton(i, c):
        sl = pl.ds(i * 16, 16)
        d = deg_sv[sl] + 1.0
        ib = plsc.bitcast(d, jnp.int32)
        y = plsc.bitcast(jnp.int32(0x5F3759DF) - (ib >> 1), jnp.float32)
        for _ in range(4):
            y = y * (1.5 - 0.5 * d * y * y)
        deg_sv[sl] = y
        iv_v[sl] = y * y
        return c
    lax.fori_loop(0, NSL // 16, _newton, 0)
    pltpu.sync_copy(deg_sv, rdeg_sh.at[sl640])

    @pl.when(cid == 0)
    def _():
        pltpu.sync_copy(iv_v, invd_out.at[sl640])
    plsc.subcore_barrier()
    pltpu.sync_copy(rdeg_sh, rdeg_v)

    if layer == 0:
        passes = [(0, dh, 5) for dh in (0, 1)]
    elif layer == 1:
        passes = [(0, dh, 10) for dh in (0, 1)]
    else:
        passes = [(p, dh, 10) for p in (0, 1) for dh in (0, 1)]

    NH = NPAD // 2

    for pi, (p, dh, n_macro) in enumerate(passes):
        if pi > 0:
            lax.fori_loop(0, 128, _zrows, 0)
            for i in range(5):
                pltpu.sync_copy(rows_b.at[pl.ds(0, 64)],
                                xs_sh.at[pl.ds(sid * 320 + i * 64, 64)])
            plsc.subcore_barrier()

        if layer == 0:
            ck = jnp.int32(0)
            base_g = cid * (EG // 2) + sid * 80
        elif layer == 1:
            ck = cid
            base_g = sid * 160
        else:
            ck = cid * 2 + p
            base_g = sid * 160

        def _phC(mi, c, base_g=base_g, ck=ck, p=p, dh=dh):
            g0 = base_g + mi * 16
            e0 = g0 * 128
            pltpu.sync_copy(src1.at[pl.ds(e0, 2048)], src_v)
            pltpu.sync_copy(dst1.at[pl.ds(e0, 2048)], dst_v)
            pltpu.sync_copy(dst2d.at[pl.ds(g0, 16)], dst_i2)
            pltpu.sync_copy(w1.at[pl.ds(e0, 2048)], w_v)

            def _lane(j, cc):
                for u in range(8):
                    sl = pl.ds(j * 128 + u * 16, 16)
                    si = src_v[sl]
                    di = dst_v[sl]
                    rs = plsc.load_gather(rdeg_v, [si])
                    rd = plsc.load_gather(rdeg_v, [di])
                    if layer == 0:
                        nrm = w_v[sl] * rs * rd
                        coef = nrm
                    else:
                        ms = plsc.load_gather(m_v, [si])
                        md = plsc.load_gather(m_v, [di])
                        cs = plsc.load_gather(c_v, [si])
                        nrm = w_v[sl] * ms * md * rs * rd
                        coef = nrm * cs
                    wp_v[sl] = nrm
                    coef_v[sl] = coef
                    gi_v[sl] = si + ck * NPAD
                    li = di - dh * NH
                    ok = (li >= 0) & (li < NH)
                    dsl_i2[j, pl.ds(u * 16, 16)] = jnp.where(
                        ok, li, NH + (di & 7))
                return cc
            lax.fori_loop(0, 16, _lane, 0)

            if p == 0 and dh == 0:
                def _ssc(j, cc):
                    pltpu.sync_copy(wp_v.at[pl.ds(j * 128, 128)],
                                    s_sh.at[dst_i2.at[j]], add=True)
                    return cc
                if layer == 0:
                    lax.fori_loop(0, 16, _ssc, 0)
                else:
                    @pl.when((sid // 8) == cid)
                    def _():
                        lax.fori_loop(0, 16, _ssc, 0)

            def _gather(j, buf, s):
                return pltpu.async_copy(
                    h_h.at[gi_v.at[pl.ds(j * 128, 128)]], buf, s)

            def _process(j, buf):
                def _row(i, cc):
                    for r in range(2):
                        l = i * 2 + r
                        cf = plsc.load_gather(
                            coef_v,
                            [jnp.zeros((16,), jnp.int32) + j * 128 + l])
                        for u in range(8):
                            sl = pl.ds(u * 16, 16)
                            buf[l, sl] = buf[l, sl] * cf
                    return cc
                lax.fori_loop(0, 64, _row, 0)
                pltpu.sync_copy(buf, xs_sh.at[dsl_i2.at[j]], add=True)

            _gather(0, rows_a, sem)

            def _pair(jj, cc):
                j0 = jj * 2
                j1 = j0 + 1
                _gather(j1, rows_b, sem2)
                pltpu.make_async_copy(
                    h_h.at[gi_v.at[pl.ds(j0 * 128, 128)]], rows_a, sem).wait()
                _process(j0, rows_a)

                @pl.when(jj < 7)
                def _():
                    _gather(j0 + 2, rows_a, sem)
                pltpu.make_async_copy(
                    h_h.at[gi_v.at[pl.ds(j1 * 128, 128)]], rows_b,
                    sem2).wait()
                _process(j1, rows_b)
                return cc
            lax.fori_loop(0, 8, _pair, 0)
            return c
        lax.fori_loop(0, n_macro, _phC, 0)
        plsc.subcore_barrier()

        if layer == 2:
            xrow = cid * 2 + p
        else:
            xrow = cid
        off = xrow * NPAD + dh * NH + sid * 320
        for i in range(5):
            pltpu.sync_copy(xs_sh.at[pl.ds(sid * 320 + i * 64, 64)],
                            xs_out.at[pl.ds(off + i * 64, 64)])
        plsc.subcore_barrier()

    pltpu.sync_copy(s_sh.at[sl640],
                    s_out.at[pl.ds(cid * NPAD + sid * NSL, NSL)])


def _make_sc_layer(layer, c_in):
    co = {0: 2, 1: 2, 2: 4}[layer]
    body = functools.partial(_sc_layer_body, layer)
    return pl.kernel(
        body,
        out_type=(
            jax.ShapeDtypeStruct((E_PAD,), jnp.float32),
            jax.ShapeDtypeStruct((NPAD,), jnp.float32),
            jax.ShapeDtypeStruct((2 * NPAD,), jnp.float32),
            jax.ShapeDtypeStruct((co * NPAD, 128), jnp.float32),
        ),
        mesh=plsc.VectorSubcoreMesh(core_axis_name="c", subcore_axis_name="s",
                                    num_cores=NCORES, num_subcores=NSUB),
        compiler_params=pltpu.CompilerParams(needs_layout_passes=False),
        scratch_types=[
            pltpu.VMEM((NPAD,), jnp.float32),
            pltpu.VMEM((NPAD,), jnp.float32),
            pltpu.VMEM((NPAD,), jnp.float32),
            pltpu.VMEM((2048,), jnp.int32),
            pltpu.VMEM((2048,), jnp.int32),
            pltpu.VMEM((2048,), jnp.float32),
            pltpu.VMEM((2048,), jnp.float32),
            pltpu.VMEM((NSL,), jnp.float32),
            pltpu.VMEM((2048,), jnp.float32),
            pltpu.VMEM((2048,), jnp.int32),
            pltpu.VMEM((16, 128), jnp.int32),
            pltpu.VMEM((16, 128), jnp.int32),
            pltpu.VMEM((128, 128), jnp.float32),
            pltpu.VMEM((128, 128), jnp.float32),
            pltpu.VMEM((NSL,), jnp.float32),
            pltpu.VMEM((NSL,), jnp.float32),
            pltpu.SemaphoreType.DMA,
            pltpu.SemaphoreType.DMA,
            pltpu.VMEM_SHARED((NPAD,), jnp.float32),
            pltpu.VMEM_SHARED((NPAD,), jnp.float32),
            pltpu.VMEM_SHARED((NPAD,), jnp.float32),
            pltpu.VMEM_SHARED((NPAD // 2 + 8, 128), jnp.float32),
        ],
    )


@functools.lru_cache(maxsize=None)
def _get_sc_layer(layer):
    return _make_sc_layer(layer, {0: 1, 1: 2, 2: 4}[layer])


def _d1_body(P, xs_ref, h_ref, sa_ref, sb_ref, iv_ref, cv_ref, m_ref,
             W_ref, b_ref, agg_ref, sum_ref, sq_ref):
    c = pl.program_id(1)
    C = pl.num_programs(1)
    rb = pl.program_id(0)
    if P == 2:
        t = xs_ref[0] + xs_ref[1]
    else:
        t = xs_ref[0]
    q = cv_ref[0, 0, :] * iv_ref[0, 0, :]
    t = t + q[:, None] * h_ref[0]
    part = jnp.dot(t, W_ref[0], preferred_element_type=jnp.float32)

    @pl.when(c == 0)
    def _():
        agg_ref[...] = part

    @pl.when(c > 0)
    def _():
        agg_ref[...] += part

    @pl.when(c == C - 1)
    def _():
        r = sa_ref[0, 0, :] + sb_ref[0, 0, :] + iv_ref[0, 0, :]
        agg = agg_ref[...] + r[:, None] * b_ref[0]
        agg_ref[...] = agg
        m = m_ref[0, 0, :]
        sm = jnp.sum(m[:, None] * agg, axis=0)[None, None, :]
        sq = jnp.sum(m[:, None] * agg * agg, axis=0)[None, None, :]

        @pl.when(rb == 0)
        def _():
            sum_ref[...] = sm
            sq_ref[...] = sq

        @pl.when(rb > 0)
        def _():
            sum_ref[...] += sm
            sq_ref[...] += sq


def _d1_call(layer, xs, h, sa, sb, iv, cv, m, W, b):
    c_in = {0: 1, 1: 2, 2: 4}[layer]
    P = 2 if layer == 0 else 1
    dout = W.shape[-1]
    if layer == 0:
        xs_spec = pl.BlockSpec((2, BN, 128), lambda rb, c: (0, rb, 0))
    else:
        xs_spec = pl.BlockSpec((1, BN, 128), lambda rb, c: (c, rb, 0))
    vec = pl.BlockSpec((1, 1, BN), lambda rb, c: (rb, 0, 0))
    return pl.pallas_call(
        functools.partial(_d1_body, P),
        grid=(NB, c_in),
        in_specs=[
            xs_spec,
            pl.BlockSpec((1, BN, 128), lambda rb, c: (c, rb, 0)),
            vec, vec, vec, vec, vec,
            pl.BlockSpec((1, 128, dout), lambda rb, c: (c, 0, 0)),
            pl.BlockSpec((1, dout), lambda rb, c: (0, 0)),
        ],
        out_specs=[
            pl.BlockSpec((BN, dout), lambda rb, c: (rb, 0)),
            pl.BlockSpec((1, 1, dout), lambda rb, c: (0, 0, 0)),
            pl.BlockSpec((1, 1, dout), lambda rb, c: (0, 0, 0)),
        ],
        out_shape=[
            jax.ShapeDtypeStruct((NPAD, dout), jnp.float32),
            jax.ShapeDtypeStruct((1, 1, dout), jnp.float32),
            jax.ShapeDtypeStruct((1, 1, dout), jnp.float32),
        ],
    )(xs, h, sa, sb, iv, cv, m, W, b)


def _d2_body(k, agg_ref, sum_ref, sq_ref, g_ref, be_ref, p_ref, h_ref, z_ref):
    c = pl.program_id(1)
    mu = sum_ref[0, 0, :] * (1.0 / k)
    ex2 = sq_ref[0, 0, :] * (1.0 / k)
    var = ex2 - mu * mu
    inv = lax.rsqrt(var + 1e-5)
    hh = (agg_ref[...] - mu[None, :]) * inv[None, :] * g_ref[0, 0, :][None, :] \
        + be_ref[0, 0, :][None, :]
    hh = jnp.maximum(hh, 0.0)
    h_ref[0] = hh
    zp = jnp.dot(hh, p_ref[0, 0, :][:, None],
                 preferred_element_type=jnp.float32)[:, 0]

    @pl.when(c == 0)
    def _():
        z_ref[...] = zp[None, None, :]

    @pl.when(c > 0)
    def _():
        z_ref[...] += zp[None, None, :]


def _d2_call(k, agg, s1, s2, g, be, p):
    dout = agg.shape[1]
    co = dout // 128
    stat = pl.BlockSpec((1, 1, 128), lambda rb, c: (0, 0, c))
    return pl.pallas_call(
        functools.partial(_d2_body, float(k)),
        grid=(NB, co),
        in_specs=[
            pl.BlockSpec((BN, 128), lambda rb, c: (rb, c)),
            stat, stat, stat, stat,
            pl.BlockSpec((1, 1, 128), lambda rb, c: (c, 0, 0)),
        ],
        out_specs=[
            pl.BlockSpec((1, BN, 128), lambda rb, c: (c, rb, 0)),
            pl.BlockSpec((1, 1, BN), lambda rb, c: (rb, 0, 0)),
        ],
        out_shape=[
            jax.ShapeDtypeStruct((co, NPAD, 128), jnp.float32),
            jax.ShapeDtypeStruct((NB, 1, BN), jnp.float32),
        ],
    )(agg, s1, s2, g, be, p)


def _d3_body(k, z_ref, m_ref, p_ref, mn_ref, cv_ref):
    z = z_ref[...]
    m = m_ref[...]
    pv = p_ref[...]
    pn = jnp.sqrt(jnp.sum(pv * pv))
    zi = lax.bitcast_convert_type(z, jnp.int32)
    key = zi ^ ((zi >> 31) & jnp.int32(0x7FFFFFFF))
    ku = lax.bitcast_convert_type(key, jnp.uint32) ^ jnp.uint32(0x80000000)
    ku = jnp.where(m > 0.0, ku, jnp.uint32(0))

    def _bit(i, T):
        cand = T | (jnp.uint32(1) << (jnp.uint32(31) - i.astype(jnp.uint32)))
        cnt = jnp.sum(jnp.where(ku >= cand, jnp.int32(1), jnp.int32(0)))
        return jnp.where(cnt >= k, cand, T)
    T = lax.fori_loop(0, 32, _bit, jnp.uint32(0))

    ngt = jnp.sum(jnp.where(ku > T, jnp.int32(1), jnp.int32(0)))
    mrem = jnp.int32(k) - ngt
    eq = ku == T
    idx = (lax.broadcasted_iota(jnp.int32, z.shape, 0) * BN
           + lax.broadcasted_iota(jnp.int32, z.shape, 2))

    def _bit2(i, Cc):
        cand = Cc | (jnp.int32(1) << (jnp.int32(13) - i))
        f = jnp.sum(jnp.where(eq & (idx < cand), jnp.int32(1), jnp.int32(0)))
        return jnp.where(f < mrem, cand, Cc)
    Cc = lax.fori_loop(0, 14, _bit2, jnp.int32(0))

    sel = (ku > T) | (eq & (idx <= Cc) & (mrem > 0))
    mn = sel.astype(jnp.float32)
    mn_ref[...] = mn
    cv_ref[...] = jnp.tanh(z * (1.0 / pn)) * mn


def _d3_call(k, z, m, p):
    d = p.shape[-1]
    return pl.pallas_call(
        functools.partial(_d3_body, k),
        in_specs=[
            pl.BlockSpec((NB, 1, BN), lambda: (0, 0, 0)),
            pl.BlockSpec((NB, 1, BN), lambda: (0, 0, 0)),
            pl.BlockSpec((1, 1, d), lambda: (0, 0, 0)),
        ],
        out_specs=[
            pl.BlockSpec((NB, 1, BN), lambda: (0, 0, 0)),
            pl.BlockSpec((NB, 1, BN), lambda: (0, 0, 0)),
        ],
        out_shape=[
            jax.ShapeDtypeStruct((NB, 1, BN), jnp.float32),
            jax.ShapeDtypeStruct((NB, 1, BN), jnp.float32),
        ],
    )(z, m, p)


def _f_body(k2, h_ref, m_ref, w2_ref, b2_ref, w1_ref, b1_ref, w0_ref, b0_ref,
            o_ref, acc_ref):
    rb = pl.program_id(0)
    m = m_ref[0, 0, :]
    s = jnp.sum(h_ref[...] * m[None, :, None], axis=1)

    @pl.when(rb == 0)
    def _():
        acc_ref[...] = s

    @pl.when(rb > 0)
    def _():
        acc_ref[...] += s

    @pl.when(rb == NB - 1)
    def _():
        a = acc_ref[...] * (1.0 / k2)
        g = jnp.zeros((1, w2_ref.shape[1]), jnp.float32)
        for j in range(8):
            g = g + jnp.dot(a[j][None, :], w2_ref[pl.ds(j * 128, 128), :],
                            preferred_element_type=jnp.float32)
        g = g + b2_ref[...]
        g = jnp.dot(g, w1_ref[...], preferred_element_type=jnp.float32) \
            + b1_ref[...]
        g = jnp.dot(g, w0_ref[...], preferred_element_type=jnp.float32) \
            + b0_ref[...]
        o_ref[...] = g


def _f_call(k2, h2, m2, L2W, L2b, L1W, L1b, L0W, L0b):
    return pl.pallas_call(
        functools.partial(_f_body, float(k2)),
        grid=(NB,),
        in_specs=[
            pl.BlockSpec((8, BN, 128), lambda rb: (0, rb, 0)),
            pl.BlockSpec((1, 1, BN), lambda rb: (rb, 0, 0)),
            pl.BlockSpec((1024, 512), lambda rb: (0, 0)),
            pl.BlockSpec((1, 512), lambda rb: (0, 0)),
            pl.BlockSpec((512, 256), lambda rb: (0, 0)),
            pl.BlockSpec((1, 256), lambda rb: (0, 0)),
            pl.BlockSpec((256, 32), lambda rb: (0, 0)),
            pl.BlockSpec((1, 32), lambda rb: (0, 0)),
        ],
        out_specs=pl.BlockSpec((1, 32), lambda rb: (0, 0)),
        out_shape=jax.ShapeDtypeStruct((1, 32), jnp.float32),
        scratch_shapes=[pltpu.VMEM((8, 128), jnp.float32)],
    )(h2, m2, L2W, L2b, L1W, L1b, L0W, L0b)


def kernel(x, edge_index, edge_attr, edge_weight, batch,
           W0, b0, g0, be0, W1, b1, g1, be1, W2, b2, g2, be2,
           p0, p1, L0W, L0b, L1W, L1b, L2W, L2b):
    f32 = jnp.float32
    src = edge_index[0]
    dst = edge_index[1]
    epad = E_PAD - E
    pad_idx = (jnp.arange(epad, dtype=jnp.int32) % N)
    src1 = jnp.concatenate([src, pad_idx])
    dst1 = jnp.concatenate([dst, pad_idx])
    w1 = jnp.concatenate([edge_weight, jnp.zeros((epad,), f32)])
    dst2d = dst1.reshape(EG, 128)
    dst1 = jnp.concatenate([dst1, jnp.zeros((8,), jnp.int32)])

    mask0 = (jnp.arange(NPAD) < N).astype(f32)
    m3 = mask0.reshape(NB, 1, BN)
    c0 = jnp.where(jnp.arange(NPAD) < N, 1.0, 0.0).astype(f32)
    x_p = jnp.pad(x, ((0, NPAD - N), (0, 0)))

    ks = [10000, 8000, 6400]
    Ws = [W0.reshape(1, 128, 256), W1.reshape(2, 128, 512),
          W2.reshape(4, 128, 1024)]
    bs = [b0.reshape(1, 256), b1.reshape(1, 512), b2.reshape(1, 1024)]
    gs = [g0.reshape(1, 1, 256), g1.reshape(1, 1, 512), g2.reshape(1, 1, 1024)]
    bes = [be0.reshape(1, 1, 256), be1.reshape(1, 1, 512),
           be2.reshape(1, 1, 1024)]
    ps = [p0, p1, None]

    m_flat = mask0
    c_flat = c0
    h_flat = x_p
    w_cur = w1
    z = None
    for layer in range(3):
        wout, invd, s2, xs = _get_sc_layer(layer)(
            src1, dst1, w_cur, dst2d, m_flat, c_flat, h_flat)
        c_in = {0: 1, 1: 2, 2: 4}[layer]
        co = {0: 2, 1: 2, 2: 4}[layer]
        xs3 = xs.reshape(co, NPAD, 128)
        h3 = h_flat.reshape(c_in, NPAD, 128)
        sa = s2[:NPAD].reshape(NB, 1, BN)
        sb = s2[NPAD:].reshape(NB, 1, BN)
        iv = invd.reshape(NB, 1, BN)
        cv3 = c_flat.reshape(NB, 1, BN)
        agg, s_sum, s_sq = _d1_call(layer, xs3, h3, sa, sb, iv, cv3, m3,
                                    Ws[layer], bs[layer])
        dout = Ws[layer].shape[2]
        pz = (ps[layer] if ps[layer] is not None
              else jnp.zeros((dout,), f32)).reshape(dout // 128, 1, 128)
        h_c, z = _d2_call(ks[layer], agg, s_sum, s_sq, gs[layer], bes[layer],
                          pz)
        if layer < 2:
            mn, cv = _d3_call(ks[layer + 1], z, m3,
                              ps[layer].reshape(1, 1, dout))
            m3 = mn
            m_flat = mn.reshape(NPAD)
            c_flat = cv.reshape(NPAD)
            h_flat = h_c.reshape(-1, 128)
            w_cur = wout
        else:
            h2 = h_c

    return _f_call(ks[2], h2, m3, L2W, L2b.reshape(1, 512),
                   L1W, L1b.reshape(1, 256), L0W, L0b.reshape(1, 32))
